# baseline (speedup 1.0000x reference)
import jax
import jax.numpy as jnp
from jax.experimental import pallas as pl
from jax.experimental.pallas import tpu as pltpu

N_OUT = 20
NC = 2
BK = 3072


def _fwd_kernel(t_ref, xt_ref, wt_ref, b_ref, o_ref, acc_ref):
    k = pl.program_id(0)
    nk = pl.num_programs(0)

    part = jax.lax.dot_general(wt_ref[...], xt_ref[...],
                               (((1,), (0,)), ((), ())),
                               preferred_element_type=jnp.float32)

    @pl.when(k == 0)
    def _init():
        acc_ref[...] = part

    @pl.when(k != 0)
    def _acc():
        acc_ref[...] += part

    @pl.when(k == nk - 1)
    def _finish():
        c0 = t_ref[0] * NC
        rows = jax.lax.broadcasted_iota(jnp.int32, o_ref.shape, 0)
        keep = (rows >= c0) & (rows < c0 + NC)
        o_ref[...] = jnp.where(keep, acc_ref[...] + b_ref[...],
                               jnp.float32(-1.0e11))


def kernel(x, t, W, b):
    B = x.shape[0]
    K = x.shape[1] * x.shape[2] * x.shape[3]
    xt = x.reshape(B, K).T
    wt = W.T
    b2 = b.reshape(N_OUT, 1)
    t_arr = jnp.asarray(t, jnp.int32).reshape((1,))
    out_t = pl.pallas_call(
        _fwd_kernel,
        grid=(K // BK,),
        in_specs=[
            pl.BlockSpec(memory_space=pltpu.SMEM),
            pl.BlockSpec((BK, B), lambda k: (k, 0)),
            pl.BlockSpec((N_OUT, BK), lambda k: (0, k)),
            pl.BlockSpec((N_OUT, 1), lambda k: (0, 0)),
        ],
        out_specs=pl.BlockSpec((N_OUT, B), lambda k: (0, 0)),
        out_shape=jax.ShapeDtypeStruct((N_OUT, B), jnp.float32),
        scratch_shapes=[pltpu.VMEM((N_OUT, B), jnp.float32)],
        compiler_params=pltpu.CompilerParams(
            dimension_semantics=("arbitrary",),
        ),
    )(t_arr, xt, wt, b2)
    return out_t.T


# speedup vs baseline: 1.0238x; 1.0238x over previous
import jax
import jax.numpy as jnp
from jax.experimental import pallas as pl
from jax.experimental.pallas import tpu as pltpu

N_OUT = 20
NC = 2
BK = 2048


def _fwd_kernel(t_ref, xt_ref, wt_ref, b_ref, o_ref, acc_ref):
    k = pl.program_id(0)
    nk = pl.num_programs(0)

    part = jax.lax.dot_general(wt_ref[...], xt_ref[...],
                               (((1,), (0,)), ((), ())),
                               preferred_element_type=jnp.float32)

    @pl.when(k == 0)
    def _init():
        acc_ref[...] = part

    @pl.when(k != 0)
    def _acc():
        acc_ref[...] += part

    @pl.when(k == nk - 1)
    def _finish():
        c0 = t_ref[0] * NC
        rows = jax.lax.broadcasted_iota(jnp.int32, o_ref.shape, 0)
        keep = (rows >= c0) & (rows < c0 + NC)
        o_ref[...] = jnp.where(keep, acc_ref[...] + b_ref[...],
                               jnp.float32(-1.0e11))


def kernel(x, t, W, b):
    B = x.shape[0]
    K = x.shape[1] * x.shape[2] * x.shape[3]
    xt = x.reshape(B, K).T
    wt = W.T
    b2 = b.reshape(N_OUT, 1)
    t_arr = jnp.asarray(t, jnp.int32).reshape((1,))
    out_t = pl.pallas_call(
        _fwd_kernel,
        grid=(K // BK,),
        in_specs=[
            pl.BlockSpec(memory_space=pltpu.SMEM),
            pl.BlockSpec((BK, B), lambda k: (k, 0)),
            pl.BlockSpec((N_OUT, BK), lambda k: (0, k)),
            pl.BlockSpec((N_OUT, 1), lambda k: (0, 0)),
        ],
        out_specs=pl.BlockSpec((N_OUT, B), lambda k: (0, 0)),
        out_shape=jax.ShapeDtypeStruct((N_OUT, B), jnp.float32),
        scratch_shapes=[pltpu.VMEM((N_OUT, B), jnp.float32)],
        compiler_params=pltpu.CompilerParams(
            dimension_semantics=("arbitrary",),
        ),
    )(t_arr, xt, wt, b2)
    return out_t.T
